# baseline (speedup 1.0000x reference)
import jax
import jax.numpy as jnp
from jax import lax
from jax.experimental import pallas as pl
from jax.experimental.pallas import tpu as pltpu
from jax.experimental.pallas import tpu_sc as plsc

B, N = 128, 32768
NW = 32
ROWS_PER_W = B // NW
CHUNK = 8192
NCH = N // CHUNK
CQ = CHUNK // 4
POS = CQ // 16
UNROLL = 4
G = ROWS_PER_W * NCH
NEG_MASK = -1000000.0
NEG_COL0 = -100000.0


def _merge(a, b):
    better = (b[0] > a[0]) | ((b[0] == a[0]) & (b[1] < a[1]))
    return (jnp.where(better, b[0], a[0]), jnp.where(better, b[1], a[1]))


def _sc_body(curr_hbm, orig_hbm, mask_hbm, frac_hbm, sel_hbm,
             cu0, cu1, og0, og1, mk0, mk1, sel_v,
             sin0, sin1, sout0, sout1):
    cid = lax.axis_index("c")
    sid = lax.axis_index("s")
    wid = sid * 2 + cid
    lanes = lax.iota(jnp.int32, 16)
    cu = (cu0, cu1)
    og = (og0, og1)
    mk = (mk0, mk1)
    sin = (sin0, sin1)
    sout = (sout0, sout1)
    mconst = []
    for j in range(4):
        v = 0xFF << (8 * j)
        if v >= 2 ** 31:
            v -= 2 ** 32
        mconst.append(jnp.full((16,), v, jnp.int32))

    def in_copies(g, b):
        row = wid * ROWS_PER_W + g // NCH
        k = g % NCH
        return (
            pltpu.make_async_copy(curr_hbm.at[row, k], cu[b], sin[b]),
            pltpu.make_async_copy(orig_hbm.at[row, k], og[b], sin[b]),
            pltpu.make_async_copy(mask_hbm.at[row, k], mk[b], sin[b]),
        )

    def out_copy(g, b):
        row = wid * ROWS_PER_W + g // NCH
        k = g % NCH
        return pltpu.make_async_copy(cu[b], frac_hbm.at[row, k], sout[b])

    def fresh_accs():
        accs = []
        for _ in range(4):
            accs.append(jnp.full((16,), -3.0e38, jnp.float32))
            accs.append(jnp.zeros((16,), jnp.int32))
        return tuple(accs)

    def compute(k, b, accs):
        cub, ogb, mkb = cu[b], og[b], mk[b]
        fix = (k == 0)

        def body(p4, carry):
            carry = list(carry)
            for u in range(UNROLL):
                p = p4 * UNROLL + u
                words = mkb[pl.ds(p * 16, 16)]
                pgv = jnp.full((16,), k * POS + p, jnp.int32)
                for j in range(4):
                    off = j * CQ + p * 16
                    c = cub[pl.ds(off, 16)]
                    o = ogb[pl.ds(off, 16)]
                    f = c / o
                    mb = words & mconst[j]
                    f = jnp.where(mb != 0, NEG_MASK, f)
                    if fix and j == 0:
                        f = jnp.where(16 * p + lanes == 0, NEG_COL0, f)
                    cub[pl.ds(off, 16)] = f
                    rm, rp = carry[2 * j], carry[2 * j + 1]
                    upd = f > rm
                    carry[2 * j] = jnp.where(upd, f, rm)
                    carry[2 * j + 1] = jnp.where(upd, pgv, rp)
            return tuple(carry)

        return lax.fori_loop(0, POS // UNROLL, body, tuple(accs))

    sel_acc = jnp.zeros((16,), jnp.int32)
    accs = fresh_accs()
    for d in in_copies(0, 0):
        d.start()
    for g in range(G):
        b = g % 2
        for d in in_copies(g, b):
            d.wait()
        if g + 1 < G:
            if g >= 1:
                out_copy(g - 1, 1 - b).wait()
            for d in in_copies(g + 1, 1 - b):
                d.start()
        accs = compute(g % NCH, b, accs)
        out_copy(g, b).start()
        if g % NCH == NCH - 1:
            r = g // NCH
            pairs = []
            for j in range(4):
                pg = accs[2 * j + 1]
                col = ((pg >> 7) << 13) + ((pg & 127) << 4) + (j * CQ) + lanes
                pairs.append((accs[2 * j], col))
            m, i = _merge(_merge(pairs[0], pairs[1]), _merge(pairs[2], pairs[3]))
            for sh in (8, 4, 2, 1):
                part = lanes ^ sh
                pm = m.at[part].get(mode="promise_in_bounds")
                pi = i.at[part].get(mode="promise_in_bounds")
                m, i = _merge((m, i), (pm, pi))
            sel_acc = jnp.where(lanes == r, i, sel_acc)
            accs = fresh_accs()
    out_copy(G - 2, G % 2).wait()
    out_copy(G - 1, 1 - G % 2).wait()
    sel_v[...] = sel_acc
    pltpu.sync_copy(sel_v, sel_hbm.at[wid])


_sc_call = pl.kernel(
    _sc_body,
    out_type=[
        jax.ShapeDtypeStruct((B, NCH, CHUNK), jnp.float32),
        jax.ShapeDtypeStruct((NW, 16), jnp.int32),
    ],
    scratch_types=[
        pltpu.VMEM((CHUNK,), jnp.float32),
        pltpu.VMEM((CHUNK,), jnp.float32),
        pltpu.VMEM((CHUNK,), jnp.float32),
        pltpu.VMEM((CHUNK,), jnp.float32),
        pltpu.VMEM((CQ,), jnp.int32),
        pltpu.VMEM((CQ,), jnp.int32),
        pltpu.VMEM((16,), jnp.int32),
        pltpu.SemaphoreType.DMA,
        pltpu.SemaphoreType.DMA,
        pltpu.SemaphoreType.DMA,
        pltpu.SemaphoreType.DMA,
    ],
    mesh=plsc.VectorSubcoreMesh(core_axis_name="c", subcore_axis_name="s"),
)


def kernel(curr_budget, orig_budget, mask):
    m8 = mask.astype(jnp.uint8).reshape(B, NCH, 4, CQ)
    m8 = jnp.transpose(m8, (0, 1, 3, 2))
    m32 = lax.bitcast_convert_type(m8, jnp.int32)
    curr3 = curr_budget.reshape(B, NCH, CHUNK)
    orig3 = orig_budget.reshape(B, NCH, CHUNK)
    frac3, sel_raw = _sc_call(curr3, orig3, m32)
    frac = frac3.reshape(B, N)
    selected = sel_raw[:, :ROWS_PER_W].reshape(B, 1)
    return frac, selected


# speedup vs baseline: 1.0224x; 1.0224x over previous
import jax
import jax.numpy as jnp
from jax import lax
from jax.experimental import pallas as pl
from jax.experimental.pallas import tpu as pltpu
from jax.experimental.pallas import tpu_sc as plsc

B, N = 128, 32768
NW = 32
ROWS_PER_W = B // NW
CHUNK = 8192
NCH = N // CHUNK
CQ = CHUNK // 4
POS = CQ // 16
UNROLL = 4
G = ROWS_PER_W * NCH
NEG_MASK = -1000000.0
NEG_COL0 = -100000.0


def _merge(a, b):
    better = (b[0] > a[0]) | ((b[0] == a[0]) & (b[1] < a[1]))
    return (jnp.where(better, b[0], a[0]), jnp.where(better, b[1], a[1]))


def _sc_body(curr_hbm, orig_hbm, mask_hbm, frac_hbm, sel_hbm,
             cu0, cu1, og0, og1, mk0, mk1, fr0, fr1, sel_v,
             sin0, sin1, sout0, sout1):
    cid = lax.axis_index("c")
    sid = lax.axis_index("s")
    wid = sid * 2 + cid
    lanes = lax.iota(jnp.int32, 16)
    cu = (cu0, cu1)
    og = (og0, og1)
    mk = (mk0, mk1)
    fr = (fr0, fr1)
    sin = (sin0, sin1)
    sout = (sout0, sout1)
    mconst = []
    for j in range(4):
        v = 0xFF << (8 * j)
        if v >= 2 ** 31:
            v -= 2 ** 32
        mconst.append(jnp.full((16,), v, jnp.int32))

    def in_copies(g, b):
        row = wid * ROWS_PER_W + g // NCH
        k = g % NCH
        return (
            pltpu.make_async_copy(curr_hbm.at[row, k], cu[b], sin[b]),
            pltpu.make_async_copy(orig_hbm.at[row, k], og[b], sin[b]),
            pltpu.make_async_copy(mask_hbm.at[row, k], mk[b], sin[b]),
        )

    def out_copy(g, b):
        row = wid * ROWS_PER_W + g // NCH
        k = g % NCH
        return pltpu.make_async_copy(fr[b], frac_hbm.at[row, k], sout[b])

    def fresh_accs():
        accs = []
        for _ in range(4):
            accs.append(jnp.full((16,), -3.0e38, jnp.float32))
            accs.append(jnp.zeros((16,), jnp.int32))
        return tuple(accs)

    def compute(k, b, accs):
        cub, ogb, mkb, frb = cu[b], og[b], mk[b], fr[b]
        fix = (k == 0)

        def body(p4, carry):
            carry = list(carry)
            for u in range(UNROLL):
                p = p4 * UNROLL + u
                words = mkb[pl.ds(p * 16, 16)]
                pgv = jnp.full((16,), k * POS + p, jnp.int32)
                for j in range(4):
                    off = j * CQ + p * 16
                    c = cub[pl.ds(off, 16)]
                    o = ogb[pl.ds(off, 16)]
                    f = c / o
                    mb = words & mconst[j]
                    f = jnp.where(mb != 0, NEG_MASK, f)
                    if fix and j == 0:
                        f = jnp.where(16 * p + lanes == 0, NEG_COL0, f)
                    frb[pl.ds(off, 16)] = f
                    rm, rp = carry[2 * j], carry[2 * j + 1]
                    upd = f > rm
                    carry[2 * j] = jnp.where(upd, f, rm)
                    carry[2 * j + 1] = jnp.where(upd, pgv, rp)
            return tuple(carry)

        return lax.fori_loop(0, POS // UNROLL, body, tuple(accs))

    sel_acc = jnp.zeros((16,), jnp.int32)
    accs = fresh_accs()
    for d in in_copies(0, 0):
        d.start()
    for g in range(G):
        b = g % 2
        for d in in_copies(g, b):
            d.wait()
        if g + 1 < G:
            for d in in_copies(g + 1, 1 - b):
                d.start()
        if g >= 2:
            out_copy(g - 2, b).wait()
        accs = compute(g % NCH, b, accs)
        out_copy(g, b).start()
        if g % NCH == NCH - 1:
            r = g // NCH
            pairs = []
            for j in range(4):
                pg = accs[2 * j + 1]
                col = ((pg >> 7) << 13) + ((pg & 127) << 4) + (j * CQ) + lanes
                pairs.append((accs[2 * j], col))
            m, i = _merge(_merge(pairs[0], pairs[1]), _merge(pairs[2], pairs[3]))
            for sh in (8, 4, 2, 1):
                part = lanes ^ sh
                pm = m.at[part].get(mode="promise_in_bounds")
                pi = i.at[part].get(mode="promise_in_bounds")
                m, i = _merge((m, i), (pm, pi))
            sel_acc = jnp.where(lanes == r, i, sel_acc)
            accs = fresh_accs()
    out_copy(G - 2, G % 2).wait()
    out_copy(G - 1, 1 - G % 2).wait()
    sel_v[...] = sel_acc
    pltpu.sync_copy(sel_v, sel_hbm.at[wid])


_sc_call = pl.kernel(
    _sc_body,
    out_type=[
        jax.ShapeDtypeStruct((B, NCH, CHUNK), jnp.float32),
        jax.ShapeDtypeStruct((NW, 16), jnp.int32),
    ],
    scratch_types=[
        pltpu.VMEM((CHUNK,), jnp.float32),
        pltpu.VMEM((CHUNK,), jnp.float32),
        pltpu.VMEM((CHUNK,), jnp.float32),
        pltpu.VMEM((CHUNK,), jnp.float32),
        pltpu.VMEM((CQ,), jnp.int32),
        pltpu.VMEM((CQ,), jnp.int32),
        pltpu.VMEM((CHUNK,), jnp.float32),
        pltpu.VMEM((CHUNK,), jnp.float32),
        pltpu.VMEM((16,), jnp.int32),
        pltpu.SemaphoreType.DMA,
        pltpu.SemaphoreType.DMA,
        pltpu.SemaphoreType.DMA,
        pltpu.SemaphoreType.DMA,
    ],
    mesh=plsc.VectorSubcoreMesh(core_axis_name="c", subcore_axis_name="s"),
)


def kernel(curr_budget, orig_budget, mask):
    m8 = mask.astype(jnp.uint8).reshape(B, NCH, 4, CQ)
    m8 = jnp.transpose(m8, (0, 1, 3, 2))
    m32 = lax.bitcast_convert_type(m8, jnp.int32)
    curr3 = curr_budget.reshape(B, NCH, CHUNK)
    orig3 = orig_budget.reshape(B, NCH, CHUNK)
    frac3, sel_raw = _sc_call(curr3, orig3, m32)
    frac = frac3.reshape(B, N)
    selected = sel_raw[:, :ROWS_PER_W].reshape(B, 1)
    return frac, selected


# speedup vs baseline: 2.2402x; 2.1912x over previous
import jax
import jax.numpy as jnp
from jax import lax
from jax.experimental import pallas as pl
from jax.experimental.pallas import tpu as pltpu
from jax.experimental.pallas import tpu_sc as plsc

B, N = 128, 32768
NW = 32
ROWS_PER_W = B // NW
CHUNK = 8192
NCH = N // CHUNK
CQ = CHUNK // 4
POS = CQ // 16
UNROLL = 4
G = ROWS_PER_W * NCH
NEG_MASK = -1000000.0
NEG_COL0 = -100000.0


def _merge(a, b):
    better = (b[0] > a[0]) | ((b[0] == a[0]) & (b[1] < a[1]))
    return (jnp.where(better, b[0], a[0]), jnp.where(better, b[1], a[1]))


def _sc_body(curr_hbm, orig_hbm, mask_hbm, frac_hbm, sel_hbm,
             cu0, cu1, og0, og1, mk0, mk1, fr0, fr1, sel_v,
             sin0, sin1, sout0, sout1):
    cid = lax.axis_index("c")
    sid = lax.axis_index("s")
    wid = sid * 2 + cid
    lanes = lax.iota(jnp.int32, 16)
    cu = (cu0, cu1)
    og = (og0, og1)
    mk = (mk0, mk1)
    fr = (fr0, fr1)
    sin = (sin0, sin1)
    sout = (sout0, sout1)
    mconst = []
    for j in range(4):
        v = 0xFF << (8 * j)
        if v >= 2 ** 31:
            v -= 2 ** 32
        mconst.append(jnp.full((16,), v, jnp.int32))

    def in_copies(g, b):
        row = wid * ROWS_PER_W + g // NCH
        k = g % NCH
        return (
            pltpu.make_async_copy(
                curr_hbm.at[row, pl.ds(k * CHUNK, CHUNK)], cu[b], sin[b]),
            pltpu.make_async_copy(
                orig_hbm.at[row, pl.ds(k * CHUNK, CHUNK)], og[b], sin[b]),
            pltpu.make_async_copy(
                mask_hbm.at[row, pl.ds(k * CQ, CQ)], mk[b], sin[b]),
        )

    def out_copy(g, b):
        row = wid * ROWS_PER_W + g // NCH
        k = g % NCH
        return pltpu.make_async_copy(
            fr[b], frac_hbm.at[row, pl.ds(k * CHUNK, CHUNK)], sout[b])

    def fresh_accs():
        accs = []
        for _ in range(4):
            accs.append(jnp.full((16,), -3.0e38, jnp.float32))
            accs.append(jnp.zeros((16,), jnp.int32))
        return tuple(accs)

    def compute(k, b, accs):
        cub, ogb, mkb, frb = cu[b], og[b], mk[b], fr[b]
        fix = (k == 0)

        @plsc.parallel_loop(0, POS // UNROLL, carry=tuple(accs), unroll=1)
        def body(p4, carry):
            carry = list(carry)
            fs = {}
            for u in range(UNROLL):
                p = p4 * UNROLL + u
                for j in range(4):
                    off = j * CQ + p * 16
                    c = cub[pl.ds(off, 16)]
                    o = ogb[pl.ds(off, 16)]
                    fs[(u, j)] = c / o
            for u in range(UNROLL):
                p = p4 * UNROLL + u
                words = mkb[pl.ds(p * 16, 16)]
                pgv = jnp.full((16,), k * POS + p, jnp.int32)
                for j in range(4):
                    off = j * CQ + p * 16
                    f = fs[(u, j)]
                    mb = words & mconst[j]
                    f = jnp.where(mb != 0, NEG_MASK, f)
                    if fix and j == 0:
                        f = jnp.where(16 * p + lanes == 0, NEG_COL0, f)
                    frb[pl.ds(off, 16)] = f
                    rm, rp = carry[2 * j], carry[2 * j + 1]
                    upd = f > rm
                    carry[2 * j] = jnp.where(upd, f, rm)
                    carry[2 * j + 1] = jnp.where(upd, pgv, rp)
            return tuple(carry)

        return body

    sel_acc = jnp.zeros((16,), jnp.int32)
    accs = fresh_accs()
    for d in in_copies(0, 0):
        d.start()
    for g in range(G):
        b = g % 2
        for d in in_copies(g, b):
            d.wait()
        if g + 1 < G:
            for d in in_copies(g + 1, 1 - b):
                d.start()
        if g >= 2:
            out_copy(g - 2, b).wait()
        accs = compute(g % NCH, b, accs)
        out_copy(g, b).start()
        if g % NCH == NCH - 1:
            r = g // NCH
            pairs = []
            for j in range(4):
                pg = accs[2 * j + 1]
                col = ((pg >> 7) << 13) + ((pg & 127) << 4) + (j * CQ) + lanes
                pairs.append((accs[2 * j], col))
            m, i = _merge(_merge(pairs[0], pairs[1]), _merge(pairs[2], pairs[3]))
            for sh in (8, 4, 2, 1):
                part = lanes ^ sh
                pm = m.at[part].get(mode="promise_in_bounds")
                pi = i.at[part].get(mode="promise_in_bounds")
                m, i = _merge((m, i), (pm, pi))
            sel_acc = jnp.where(lanes == r, i, sel_acc)
            accs = fresh_accs()
    out_copy(G - 2, G % 2).wait()
    out_copy(G - 1, 1 - G % 2).wait()
    sel_v[...] = sel_acc
    pltpu.sync_copy(sel_v, sel_hbm.at[wid])


_sc_call = pl.kernel(
    _sc_body,
    out_type=[
        jax.ShapeDtypeStruct((B, N), jnp.float32),
        jax.ShapeDtypeStruct((NW, 16), jnp.int32),
    ],
    scratch_types=[
        pltpu.VMEM((CHUNK,), jnp.float32),
        pltpu.VMEM((CHUNK,), jnp.float32),
        pltpu.VMEM((CHUNK,), jnp.float32),
        pltpu.VMEM((CHUNK,), jnp.float32),
        pltpu.VMEM((CQ,), jnp.int32),
        pltpu.VMEM((CQ,), jnp.int32),
        pltpu.VMEM((CHUNK,), jnp.float32),
        pltpu.VMEM((CHUNK,), jnp.float32),
        pltpu.VMEM((16,), jnp.int32),
        pltpu.SemaphoreType.DMA,
        pltpu.SemaphoreType.DMA,
        pltpu.SemaphoreType.DMA,
        pltpu.SemaphoreType.DMA,
    ],
    mesh=plsc.VectorSubcoreMesh(core_axis_name="c", subcore_axis_name="s"),
)


def kernel(curr_budget, orig_budget, mask):
    m8 = mask.astype(jnp.uint8).reshape(B, NCH, 4, CQ)
    m8 = jnp.transpose(m8, (0, 1, 3, 2))
    m32 = lax.bitcast_convert_type(m8, jnp.int32).reshape(B, NCH * CQ)
    frac, sel_raw = _sc_call(curr_budget, orig_budget, m32)
    selected = sel_raw[:, :ROWS_PER_W].reshape(B, 1)
    return frac, selected
